# baseline (speedup 1.0000x reference)
import functools

import jax
import jax.numpy as jnp
from jax import lax
from jax.experimental import pallas as pl
from jax.experimental.pallas import tpu as pltpu
from jax.experimental.pallas import tpu_sc as plsc

N_DAYS = 2048
N_BUCKETS = 128
BATCH = 16384
MVOC_LO = 0.0
MVOC_HI = 1.0
MVOC_MEAN = 0.45

_NUM_CORES = 2
_NUM_SUBCORES = 16
_NW = _NUM_CORES * _NUM_SUBCORES
_BPW = BATCH // _NW
_L = 16


_NCHUNK = 4
_CSZ = _BPW // _NCHUNK


def _sc_body(mvoc_hbm, day_hbm, bkt_hbm, shift_hbm, tilt_hbm, out_hbm,
             idx_v, bkt_v, mv_v, s_v, t_v, out_v,
             sem_g, sem_in, sem_m, sem_w):
    wid = lax.axis_index("s") * _NUM_CORES + lax.axis_index("c")
    base = wid * _BPW

    cp_d = pltpu.async_copy(day_hbm.at[pl.ds(base, _BPW)], idx_v, sem_in)
    cp_b = pltpu.async_copy(bkt_hbm.at[pl.ds(base, _BPW)], bkt_v, sem_in)
    cp_m = pltpu.async_copy(mvoc_hbm.at[pl.ds(base, _BPW)], mv_v, sem_m)
    cp_d.wait()
    cp_b.wait()

    gathers = []
    for c in range(_NCHUNK):
        for i in range(_CSZ // _L):
            off = c * _CSZ + i * _L
            idx_v[pl.ds(off, _L)] = (
                idx_v[pl.ds(off, _L)] * N_BUCKETS + bkt_v[pl.ds(off, _L)])
        ci = idx_v.at[pl.ds(c * _CSZ, _CSZ)]
        cp_s = pltpu.async_copy(
            shift_hbm.at[ci], s_v.at[pl.ds(c * _CSZ, _CSZ)], sem_g.at[c])
        cp_t = pltpu.async_copy(
            tilt_hbm.at[ci], t_v.at[pl.ds(c * _CSZ, _CSZ)], sem_g.at[c])
        gathers.append((cp_s, cp_t))

    span = max(MVOC_HI - MVOC_LO, 1e-12)
    z_bar = jnp.float32((MVOC_MEAN - MVOC_LO) / span)
    inv_span = jnp.float32(1.0 / span)
    lo = jnp.float32(MVOC_LO)

    cp_m.wait()
    outs = []
    for c in range(_NCHUNK):
        cp_s, cp_t = gathers[c]
        cp_s.wait()
        cp_t.wait()
        for i in range(_CSZ // _L):
            off = c * _CSZ + i * _L
            z = jnp.clip((mv_v[pl.ds(off, _L)] - lo) * inv_span, 0.0, 1.0)
            out_v[pl.ds(off, _L)] = (
                s_v[pl.ds(off, _L)] + t_v[pl.ds(off, _L)] * (z_bar - z))
        outs.append(pltpu.async_copy(
            out_v.at[pl.ds(c * _CSZ, _CSZ)],
            out_hbm.at[pl.ds(base + c * _CSZ, _CSZ)], sem_w))
    for cp in outs:
        cp.wait()


@functools.partial(jax.jit, static_argnames=())
def _run(mvoc, day_idx, bucket_idx, shift_flat, tilt_flat):
    mesh = plsc.VectorSubcoreMesh(core_axis_name="c", subcore_axis_name="s")
    return pl.kernel(
        _sc_body,
        out_type=jax.ShapeDtypeStruct((BATCH,), jnp.float32),
        mesh=mesh,
        scratch_types=[
            pltpu.VMEM((_BPW,), jnp.int32),
            pltpu.VMEM((_BPW,), jnp.int32),
            pltpu.VMEM((_BPW,), jnp.float32),
            pltpu.VMEM((_BPW,), jnp.float32),
            pltpu.VMEM((_BPW,), jnp.float32),
            pltpu.VMEM((_BPW,), jnp.float32),
            pltpu.SemaphoreType.DMA((_NCHUNK,)),
            pltpu.SemaphoreType.DMA,
            pltpu.SemaphoreType.DMA,
            pltpu.SemaphoreType.DMA,
        ],
    )(mvoc, day_idx, bucket_idx, shift_flat, tilt_flat)


def kernel(mvoc, day_idx, bucket_idx, shift, tilt):
    out = _run(
        mvoc.reshape(-1),
        day_idx.reshape(-1),
        bucket_idx.reshape(-1),
        shift.reshape(-1),
        tilt.reshape(-1),
    )
    return out.reshape(-1, 1)


# speedup vs baseline: 1.0010x; 1.0010x over previous
import functools

import jax
import jax.numpy as jnp
from jax import lax
from jax.experimental import pallas as pl
from jax.experimental.pallas import tpu as pltpu
from jax.experimental.pallas import tpu_sc as plsc

N_DAYS = 2048
N_BUCKETS = 128
BATCH = 16384
MVOC_LO = 0.0
MVOC_HI = 1.0
MVOC_MEAN = 0.45

_NUM_CORES = 2
_NUM_SUBCORES = 16
_NW = _NUM_CORES * _NUM_SUBCORES
_BPW = BATCH // _NW
_L = 16


_NCHUNK = 4
_CSZ = _BPW // _NCHUNK


def _sc_body(mvoc_hbm, day_hbm, bkt_hbm, shift_hbm, tilt_hbm, out_hbm,
             idx_v, bkt_v, mv_v, s_v, t_v, out_v,
             sem_g, sem_in, sem_m, sem_w):
    wid = lax.axis_index("s") * _NUM_CORES + lax.axis_index("c")
    base = wid * _BPW

    cp_d = pltpu.async_copy(day_hbm.at[pl.ds(base, _BPW)], idx_v, sem_in)
    cp_b = pltpu.async_copy(bkt_hbm.at[pl.ds(base, _BPW)], bkt_v, sem_in)
    cp_m = pltpu.async_copy(mvoc_hbm.at[pl.ds(base, _BPW)], mv_v, sem_m)
    cp_d.wait()
    cp_b.wait()

    gathers = []
    for c in range(_NCHUNK):
        def _idx_step(i, _, c=c):
            off = c * _CSZ + i * _L
            idx_v[pl.ds(off, _L)] = (
                idx_v[pl.ds(off, _L)] * N_BUCKETS + bkt_v[pl.ds(off, _L)])
            return _
        lax.fori_loop(0, _CSZ // _L, _idx_step, 0)
        ci = idx_v.at[pl.ds(c * _CSZ, _CSZ)]
        cp_s = pltpu.async_copy(
            shift_hbm.at[ci], s_v.at[pl.ds(c * _CSZ, _CSZ)], sem_g.at[c])
        cp_t = pltpu.async_copy(
            tilt_hbm.at[ci], t_v.at[pl.ds(c * _CSZ, _CSZ)], sem_g.at[c])
        gathers.append((cp_s, cp_t))

    span = max(MVOC_HI - MVOC_LO, 1e-12)
    z_bar = jnp.float32((MVOC_MEAN - MVOC_LO) / span)
    inv_span = jnp.float32(1.0 / span)
    lo = jnp.float32(MVOC_LO)

    cp_m.wait()
    outs = []
    for c in range(_NCHUNK):
        cp_s, cp_t = gathers[c]
        cp_s.wait()
        cp_t.wait()
        def _out_step(i, _, c=c):
            off = c * _CSZ + i * _L
            z = jnp.clip((mv_v[pl.ds(off, _L)] - lo) * inv_span, 0.0, 1.0)
            out_v[pl.ds(off, _L)] = (
                s_v[pl.ds(off, _L)] + t_v[pl.ds(off, _L)] * (z_bar - z))
            return _
        lax.fori_loop(0, _CSZ // _L, _out_step, 0)
        outs.append(pltpu.async_copy(
            out_v.at[pl.ds(c * _CSZ, _CSZ)],
            out_hbm.at[pl.ds(base + c * _CSZ, _CSZ)], sem_w))
    for cp in outs:
        cp.wait()


@functools.partial(jax.jit, static_argnames=())
def _run(mvoc, day_idx, bucket_idx, shift_flat, tilt_flat):
    mesh = plsc.VectorSubcoreMesh(core_axis_name="c", subcore_axis_name="s")
    return pl.kernel(
        _sc_body,
        out_type=jax.ShapeDtypeStruct((BATCH,), jnp.float32),
        mesh=mesh,
        scratch_types=[
            pltpu.VMEM((_BPW,), jnp.int32),
            pltpu.VMEM((_BPW,), jnp.int32),
            pltpu.VMEM((_BPW,), jnp.float32),
            pltpu.VMEM((_BPW,), jnp.float32),
            pltpu.VMEM((_BPW,), jnp.float32),
            pltpu.VMEM((_BPW,), jnp.float32),
            pltpu.SemaphoreType.DMA((_NCHUNK,)),
            pltpu.SemaphoreType.DMA,
            pltpu.SemaphoreType.DMA,
            pltpu.SemaphoreType.DMA,
        ],
    )(mvoc, day_idx, bucket_idx, shift_flat, tilt_flat)


def kernel(mvoc, day_idx, bucket_idx, shift, tilt):
    out = _run(
        mvoc.reshape(-1),
        day_idx.reshape(-1),
        bucket_idx.reshape(-1),
        shift.reshape(-1),
        tilt.reshape(-1),
    )
    return out.reshape(-1, 1)


# speedup vs baseline: 1.0017x; 1.0007x over previous
import functools

import jax
import jax.numpy as jnp
from jax import lax
from jax.experimental import pallas as pl
from jax.experimental.pallas import tpu as pltpu
from jax.experimental.pallas import tpu_sc as plsc

N_DAYS = 2048
N_BUCKETS = 128
BATCH = 16384
MVOC_LO = 0.0
MVOC_HI = 1.0
MVOC_MEAN = 0.45

_NUM_CORES = 2
_NUM_SUBCORES = 16
_NW = _NUM_CORES * _NUM_SUBCORES
_BPW = BATCH // _NW
_L = 16


_NCHUNK = 4
_CSZ = _BPW // _NCHUNK


def _sc_body(mvoc_hbm, day_hbm, bkt_hbm, shift_hbm, tilt_hbm, out_hbm,
             iv, fv, sems):
    wid = lax.axis_index("s") * _NUM_CORES + lax.axis_index("c")
    base = wid * _BPW

    idx_v = iv.at[pl.ds(0, _BPW)]
    bkt_v = iv.at[pl.ds(_BPW, _BPW)]
    mv_v = fv.at[pl.ds(0, _BPW)]
    s_v = fv.at[pl.ds(_BPW, _BPW)]
    t_v = fv.at[pl.ds(2 * _BPW, _BPW)]
    out_v = fv.at[pl.ds(3 * _BPW, _BPW)]
    sem_in = sems.at[_NCHUNK]
    sem_m = sems.at[_NCHUNK + 1]
    sem_w = sems.at[_NCHUNK + 2]

    cp_d = pltpu.async_copy(day_hbm.at[pl.ds(base, _BPW)], idx_v, sem_in)
    cp_b = pltpu.async_copy(bkt_hbm.at[pl.ds(base, _BPW)], bkt_v, sem_in)
    cp_m = pltpu.async_copy(mvoc_hbm.at[pl.ds(base, _BPW)], mv_v, sem_m)
    cp_d.wait()
    cp_b.wait()

    gathers = []
    for c in range(_NCHUNK):
        def _idx_step(i, _, c=c):
            off = c * _CSZ + i * _L
            idx_v[pl.ds(off, _L)] = (
                idx_v[pl.ds(off, _L)] * N_BUCKETS + bkt_v[pl.ds(off, _L)])
            return _
        lax.fori_loop(0, _CSZ // _L, _idx_step, 0)
        ci = idx_v.at[pl.ds(c * _CSZ, _CSZ)]
        cp_s = pltpu.async_copy(
            shift_hbm.at[ci], s_v.at[pl.ds(c * _CSZ, _CSZ)], sems.at[c])
        cp_t = pltpu.async_copy(
            tilt_hbm.at[ci], t_v.at[pl.ds(c * _CSZ, _CSZ)], sems.at[c])
        gathers.append((cp_s, cp_t))

    span = max(MVOC_HI - MVOC_LO, 1e-12)
    z_bar = jnp.float32((MVOC_MEAN - MVOC_LO) / span)
    inv_span = jnp.float32(1.0 / span)
    lo = jnp.float32(MVOC_LO)

    cp_m.wait()
    outs = []
    for c in range(_NCHUNK):
        cp_s, cp_t = gathers[c]
        cp_s.wait()
        cp_t.wait()
        def _out_step(i, _, c=c):
            off = c * _CSZ + i * _L
            z = jnp.clip((mv_v[pl.ds(off, _L)] - lo) * inv_span, 0.0, 1.0)
            out_v[pl.ds(off, _L)] = (
                s_v[pl.ds(off, _L)] + t_v[pl.ds(off, _L)] * (z_bar - z))
            return _
        lax.fori_loop(0, _CSZ // _L, _out_step, 0)
        outs.append(pltpu.async_copy(
            out_v.at[pl.ds(c * _CSZ, _CSZ)],
            out_hbm.at[pl.ds(base + c * _CSZ, _CSZ)], sem_w))
    for cp in outs:
        cp.wait()


@functools.partial(jax.jit, static_argnames=())
def _run(mvoc, day_idx, bucket_idx, shift_flat, tilt_flat):
    mesh = plsc.VectorSubcoreMesh(core_axis_name="c", subcore_axis_name="s")
    return pl.kernel(
        _sc_body,
        out_type=jax.ShapeDtypeStruct((BATCH,), jnp.float32),
        mesh=mesh,
        scratch_types=[
            pltpu.VMEM((2 * _BPW,), jnp.int32),
            pltpu.VMEM((4 * _BPW,), jnp.float32),
            pltpu.SemaphoreType.DMA((_NCHUNK + 3,)),
        ],
    )(mvoc, day_idx, bucket_idx, shift_flat, tilt_flat)


def kernel(mvoc, day_idx, bucket_idx, shift, tilt):
    out = _run(
        mvoc.reshape(-1),
        day_idx.reshape(-1),
        bucket_idx.reshape(-1),
        shift.reshape(-1),
        tilt.reshape(-1),
    )
    return out.reshape(-1, 1)
